# baseline (speedup 1.0000x reference)
import jax
import jax.numpy as jnp
from jax import lax
from jax.experimental import pallas as pl
from jax.experimental.pallas import tpu as pltpu
from jax.experimental.pallas import tpu_sc as plsc

_B = 16384
_K = 32
_P = _K // 2
_N = 1000000
_NW = 32
_BPW = _B // _NW
_CHUNK = 128
_NCHUNK = _BPW // _CHUNK
_W = 65536

_mesh = plsc.VectorSubcoreMesh(core_axis_name="c", subcore_axis_name="s")


def _slab_body(in_ref, *outs):
    for j in range(_P):
        lo = lax.bitcast_convert_type(
            in_ref[2 * j, :].astype(jnp.bfloat16), jnp.uint16
        ).astype(jnp.uint32)
        hi = lax.bitcast_convert_type(
            in_ref[2 * j + 1, :].astype(jnp.bfloat16), jnp.uint16
        ).astype(jnp.uint32)
        outs[j][...] = lax.bitcast_convert_type(lo | (hi << 16), jnp.int32)


_slab = pl.pallas_call(
    _slab_body,
    grid=((_N + _W - 1) // _W,),
    in_specs=[pl.BlockSpec((_K, _W), lambda i: (0, i))],
    out_specs=[pl.BlockSpec((_W,), lambda i: (i,))] * _P,
    out_shape=[jax.ShapeDtypeStruct((_N,), jnp.int32)] * _P,
)


def _bias_body(in_ref, out_ref):
    out_ref[...] = in_ref[0, :]


_bias_flat = pl.pallas_call(
    _bias_body,
    grid=((_N + _W - 1) // _W,),
    in_specs=[pl.BlockSpec((1, _W), lambda i: (0, i))],
    out_specs=pl.BlockSpec((_W,), lambda i: (i,)),
    out_shape=jax.ShapeDtypeStruct((_N,), jnp.float32),
)


def _mf_body(uid_hbm, iid_hbm, *refs):
    u_slabs = refs[:_P]
    v_slabs = refs[_P:2 * _P]
    bf_hbm = refs[2 * _P]
    out_hbm = refs[2 * _P + 1]
    uid_v, iid_v, u_flat, v_flat, b_flat, out_v, sem, bsem = refs[2 * _P + 2:]

    wid = lax.axis_index("s") * 2 + lax.axis_index("c")
    base = wid * _BPW

    for c in range(_NCHUNK):
        pltpu.sync_copy(uid_hbm.at[pl.ds(base + c * _CHUNK, _CHUNK)],
                        uid_v.at[c])
        pltpu.sync_copy(iid_hbm.at[pl.ds(base + c * _CHUNK, _CHUNK)],
                        iid_v.at[c])

    bias_copies = [
        pltpu.async_copy(bf_hbm.at[iid_v.at[c]],
                         b_flat.at[pl.ds(c * _CHUNK, _CHUNK)], bsem)
        for c in range(_NCHUNK)
    ]

    for c in range(_NCHUNK):
        for j in range(_P):
            pltpu.async_copy(
                u_slabs[j].at[uid_v.at[c]],
                u_flat.at[pl.ds(j * _BPW + c * _CHUNK, _CHUNK)], sem)
            pltpu.async_copy(
                v_slabs[j].at[iid_v.at[c]],
                v_flat.at[pl.ds(j * _BPW + c * _CHUNK, _CHUNK)], sem)

    pltpu.make_async_copy(uid_hbm.at[pl.ds(0, _BPW * _P)], u_flat, sem).wait()
    pltpu.make_async_copy(uid_hbm.at[pl.ds(0, _BPW * _P)], v_flat, sem).wait()
    for cp in bias_copies:
        cp.wait()

    lo_mask = jnp.full((16,), 0xFFFF, jnp.int32)
    hi_mask = lax.bitcast_convert_type(
        jnp.full((16,), 0xFFFF0000, jnp.uint32), jnp.int32)

    def group(g, carry):
        acc0 = b_flat[pl.ds(g * 16, 16)]
        acc1 = jnp.zeros((16,), jnp.float32)
        acc2 = jnp.zeros((16,), jnp.float32)
        acc3 = jnp.zeros((16,), jnp.float32)
        accs = [acc0, acc1, acc2, acc3]
        for j in range(_P):
            up = u_flat[pl.ds(j * _BPW + g * 16, 16)]
            vp = v_flat[pl.ds(j * _BPW + g * 16, 16)]
            u_lo = lax.bitcast_convert_type(
                lax.shift_left(up, jnp.full((16,), 16, jnp.int32)),
                jnp.float32)
            v_lo = lax.bitcast_convert_type(
                lax.shift_left(vp, jnp.full((16,), 16, jnp.int32)),
                jnp.float32)
            u_hi = lax.bitcast_convert_type(up & hi_mask, jnp.float32)
            v_hi = lax.bitcast_convert_type(vp & hi_mask, jnp.float32)
            accs[(2 * j) % 4] = accs[(2 * j) % 4] + u_lo * v_lo
            accs[(2 * j + 1) % 4] = accs[(2 * j + 1) % 4] + u_hi * v_hi
        out_v[pl.ds(g * 16, 16)] = (accs[0] + accs[1]) + (accs[2] + accs[3])
        return carry

    lax.fori_loop(0, _BPW // 16, group, 0)

    pltpu.sync_copy(out_v, out_hbm.at[pl.ds(base, _BPW)])


_mf_kernel = pl.kernel(
    _mf_body,
    out_type=jax.ShapeDtypeStruct((_B,), jnp.float32),
    mesh=_mesh,
    scratch_types=[
        pltpu.VMEM((_NCHUNK, _CHUNK), jnp.int32),
        pltpu.VMEM((_NCHUNK, _CHUNK), jnp.int32),
        pltpu.VMEM((_BPW * _P,), jnp.int32),
        pltpu.VMEM((_BPW * _P,), jnp.int32),
        pltpu.VMEM((_BPW,), jnp.float32),
        pltpu.VMEM((_BPW,), jnp.float32),
        pltpu.SemaphoreType.DMA,
        pltpu.SemaphoreType.DMA,
    ],
    compiler_params=pltpu.CompilerParams(needs_layout_passes=False),
)


def kernel(user_ids, item_ids, user_factors, item_factors, bias_factors):
    uids = user_ids.astype(jnp.int32)
    iids = item_ids.astype(jnp.int32)
    u_slabs = _slab(user_factors.T)
    v_slabs = _slab(item_factors.T)
    bias = _bias_flat(bias_factors.T)
    return _mf_kernel(uids, iids, *u_slabs, *v_slabs, bias)


# speedup vs baseline: 2.0728x; 2.0728x over previous
import jax
import jax.numpy as jnp
from jax import lax
from jax.experimental import pallas as pl
from jax.experimental.pallas import tpu as pltpu
from jax.experimental.pallas import tpu_sc as plsc

_B = 16384
_K = 32
_N = 1000000
_NW = 32
_BPW = _B // _NW
_CHUNK = 128
_NCHUNK = _BPW // _CHUNK
_W = 98304

_mesh = plsc.VectorSubcoreMesh(core_axis_name="c", subcore_axis_name="s")


def _slab_body(in_ref, *outs):
    for k in range(_K):
        outs[k][...] = in_ref[k, :]


_slab = pl.pallas_call(
    _slab_body,
    grid=((_N + _W - 1) // _W,),
    in_specs=[pl.BlockSpec((_K, _W), lambda i: (0, i))],
    out_specs=[pl.BlockSpec((_W,), lambda i: (i,))] * _K,
    out_shape=[jax.ShapeDtypeStruct((_N,), jnp.float32)] * _K,
)


def _bias_body(in_ref, out_ref):
    out_ref[...] = in_ref[0, :]


_bias_flat = pl.pallas_call(
    _bias_body,
    grid=((_N + _W - 1) // _W,),
    in_specs=[pl.BlockSpec((1, _W), lambda i: (0, i))],
    out_specs=pl.BlockSpec((_W,), lambda i: (i,)),
    out_shape=jax.ShapeDtypeStruct((_N,), jnp.float32),
)


def _mf_body(uid_hbm, iid_hbm, *refs):
    u_slabs = refs[:_K]
    v_slabs = refs[_K:2 * _K]
    bf_hbm = refs[2 * _K]
    out_hbm = refs[2 * _K + 1]
    uid_v, iid_v, u_flat, v_flat, b_flat, out_v, sem, bsem = refs[2 * _K + 2:]

    wid = lax.axis_index("s") * 2 + lax.axis_index("c")
    base = wid * _BPW

    for c in range(_NCHUNK):
        pltpu.sync_copy(uid_hbm.at[pl.ds(base + c * _CHUNK, _CHUNK)],
                        uid_v.at[c])
        pltpu.sync_copy(iid_hbm.at[pl.ds(base + c * _CHUNK, _CHUNK)],
                        iid_v.at[c])

    bias_copies = [
        pltpu.async_copy(bf_hbm.at[iid_v.at[c]],
                         b_flat.at[pl.ds(c * _CHUNK, _CHUNK)], bsem)
        for c in range(_NCHUNK)
    ]

    for c in range(_NCHUNK):
        for k in range(_K):
            pltpu.async_copy(
                u_slabs[k].at[uid_v.at[c]],
                u_flat.at[pl.ds(k * _BPW + c * _CHUNK, _CHUNK)], sem)
            pltpu.async_copy(
                v_slabs[k].at[iid_v.at[c]],
                v_flat.at[pl.ds(k * _BPW + c * _CHUNK, _CHUNK)], sem)

    pltpu.make_async_copy(bf_hbm.at[pl.ds(0, _BPW * _K)], u_flat, sem).wait()
    pltpu.make_async_copy(bf_hbm.at[pl.ds(0, _BPW * _K)], v_flat, sem).wait()
    for cp in bias_copies:
        cp.wait()

    def group(g, carry):
        acc0 = b_flat[pl.ds(g * 16, 16)]
        acc1 = jnp.zeros((16,), jnp.float32)
        acc2 = jnp.zeros((16,), jnp.float32)
        acc3 = jnp.zeros((16,), jnp.float32)
        accs = [acc0, acc1, acc2, acc3]
        for k in range(_K):
            u = u_flat[pl.ds(k * _BPW + g * 16, 16)]
            v = v_flat[pl.ds(k * _BPW + g * 16, 16)]
            accs[k % 4] = accs[k % 4] + u * v
        out_v[pl.ds(g * 16, 16)] = (accs[0] + accs[1]) + (accs[2] + accs[3])
        return carry

    lax.fori_loop(0, _BPW // 16, group, 0)

    pltpu.sync_copy(out_v, out_hbm.at[pl.ds(base, _BPW)])


_mf_kernel = pl.kernel(
    _mf_body,
    out_type=jax.ShapeDtypeStruct((_B,), jnp.float32),
    mesh=_mesh,
    scratch_types=[
        pltpu.VMEM((_NCHUNK, _CHUNK), jnp.int32),
        pltpu.VMEM((_NCHUNK, _CHUNK), jnp.int32),
        pltpu.VMEM((_BPW * _K,), jnp.float32),
        pltpu.VMEM((_BPW * _K,), jnp.float32),
        pltpu.VMEM((_BPW,), jnp.float32),
        pltpu.VMEM((_BPW,), jnp.float32),
        pltpu.SemaphoreType.DMA,
        pltpu.SemaphoreType.DMA,
    ],
    compiler_params=pltpu.CompilerParams(needs_layout_passes=False),
)


def kernel(user_ids, item_ids, user_factors, item_factors, bias_factors):
    uids = user_ids.astype(jnp.int32)
    iids = item_ids.astype(jnp.int32)
    u_slabs = _slab(user_factors.T)
    v_slabs = _slab(item_factors.T)
    bias = _bias_flat(bias_factors.T)
    return _mf_kernel(uids, iids, *u_slabs, *v_slabs, bias)


# speedup vs baseline: 2.1630x; 1.0435x over previous
import jax
import jax.numpy as jnp
from jax import lax
from jax.experimental import pallas as pl
from jax.experimental.pallas import tpu as pltpu
from jax.experimental.pallas import tpu_sc as plsc

_B = 16384
_K = 32
_N = 1000000
_NW = 32
_BPW = _B // _NW
_CHUNK = 128
_NCHUNK = _BPW // _CHUNK
_W = 65536

_mesh = plsc.VectorSubcoreMesh(core_axis_name="c", subcore_axis_name="s")


def _slab_body(in_ref, *outs):
    for k in range(_K):
        outs[k][...] = in_ref[k, :]


_slab = pl.pallas_call(
    _slab_body,
    grid=((_N + _W - 1) // _W,),
    in_specs=[pl.BlockSpec((_K, _W), lambda i: (0, i))],
    out_specs=[pl.BlockSpec((_W,), lambda i: (i,))] * _K,
    out_shape=[jax.ShapeDtypeStruct((_N,), jnp.float32)] * _K,
)


def _bias_body(in_ref, out_ref):
    out_ref[...] = in_ref[0, :]


_bias_flat = pl.pallas_call(
    _bias_body,
    grid=((_N + _W - 1) // _W,),
    in_specs=[pl.BlockSpec((1, _W), lambda i: (0, i))],
    out_specs=pl.BlockSpec((_W,), lambda i: (i,)),
    out_shape=jax.ShapeDtypeStruct((_N,), jnp.float32),
)


def _gather_u_body(uid_hbm, *refs):
    u_slabs = refs[:_K]
    ug_hbm = refs[_K]
    uid_v, u_flat, sem = refs[_K + 1:]

    wid = lax.axis_index("s") * 2 + lax.axis_index("c")
    base = wid * _BPW

    for c in range(_NCHUNK):
        pltpu.sync_copy(uid_hbm.at[pl.ds(base + c * _CHUNK, _CHUNK)],
                        uid_v.at[c])
    for c in range(_NCHUNK):
        for k in range(_K):
            pltpu.async_copy(
                u_slabs[k].at[uid_v.at[c]],
                u_flat.at[pl.ds(k * _BPW + c * _CHUNK, _CHUNK)], sem)
    pltpu.make_async_copy(
        u_slabs[0].at[pl.ds(0, _BPW * _K)], u_flat, sem).wait()
    pltpu.sync_copy(u_flat, ug_hbm.at[pl.ds(wid * _BPW * _K, _BPW * _K)])


_gather_u = pl.kernel(
    _gather_u_body,
    out_type=jax.ShapeDtypeStruct((_B * _K,), jnp.float32),
    mesh=_mesh,
    scratch_types=[
        pltpu.VMEM((_NCHUNK, _CHUNK), jnp.int32),
        pltpu.VMEM((_BPW * _K,), jnp.float32),
        pltpu.SemaphoreType.DMA,
    ],
    compiler_params=pltpu.CompilerParams(needs_layout_passes=False),
)


def _mf_body(iid_hbm, ug_hbm, *refs):
    v_slabs = refs[:_K]
    bf_hbm = refs[_K]
    out_hbm = refs[_K + 1]
    iid_v, u_flat, v_flat, b_flat, out_v, sem, bsem = refs[_K + 2:]

    wid = lax.axis_index("s") * 2 + lax.axis_index("c")
    base = wid * _BPW

    for c in range(_NCHUNK):
        pltpu.sync_copy(iid_hbm.at[pl.ds(base + c * _CHUNK, _CHUNK)],
                        iid_v.at[c])

    bias_copies = [
        pltpu.async_copy(bf_hbm.at[iid_v.at[c]],
                         b_flat.at[pl.ds(c * _CHUNK, _CHUNK)], bsem)
        for c in range(_NCHUNK)
    ]

    ucp = pltpu.async_copy(
        ug_hbm.at[pl.ds(wid * _BPW * _K, _BPW * _K)], u_flat, bsem)
    for c in range(_NCHUNK):
        for k in range(_K):
            pltpu.async_copy(
                v_slabs[k].at[iid_v.at[c]],
                v_flat.at[pl.ds(k * _BPW + c * _CHUNK, _CHUNK)], sem)

    pltpu.make_async_copy(ug_hbm.at[pl.ds(0, _BPW * _K)], v_flat, sem).wait()
    ucp.wait()
    for cp in bias_copies:
        cp.wait()

    def group(g, carry):
        acc0 = b_flat[pl.ds(g * 16, 16)]
        acc1 = jnp.zeros((16,), jnp.float32)
        acc2 = jnp.zeros((16,), jnp.float32)
        acc3 = jnp.zeros((16,), jnp.float32)
        accs = [acc0, acc1, acc2, acc3]
        for k in range(_K):
            u = u_flat[pl.ds(k * _BPW + g * 16, 16)]
            v = v_flat[pl.ds(k * _BPW + g * 16, 16)]
            accs[k % 4] = accs[k % 4] + u * v
        out_v[pl.ds(g * 16, 16)] = (accs[0] + accs[1]) + (accs[2] + accs[3])
        return carry

    lax.fori_loop(0, _BPW // 16, group, 0)

    pltpu.sync_copy(out_v, out_hbm.at[pl.ds(base, _BPW)])


_mf_kernel = pl.kernel(
    _mf_body,
    out_type=jax.ShapeDtypeStruct((_B,), jnp.float32),
    mesh=_mesh,
    scratch_types=[
        pltpu.VMEM((_NCHUNK, _CHUNK), jnp.int32),
        pltpu.VMEM((_BPW * _K,), jnp.float32),
        pltpu.VMEM((_BPW * _K,), jnp.float32),
        pltpu.VMEM((_BPW,), jnp.float32),
        pltpu.VMEM((_BPW,), jnp.float32),
        pltpu.SemaphoreType.DMA,
        pltpu.SemaphoreType.DMA,
    ],
    compiler_params=pltpu.CompilerParams(needs_layout_passes=False),
)


def kernel(user_ids, item_ids, user_factors, item_factors, bias_factors):
    uids = user_ids.astype(jnp.int32)
    iids = item_ids.astype(jnp.int32)
    u_slabs = _slab(user_factors.T)
    u_g = _gather_u(uids, *u_slabs)
    v_slabs = _slab(item_factors.T)
    bias = _bias_flat(bias_factors.T)
    return _mf_kernel(iids, u_g, *v_slabs, bias)
